# baseline (speedup 1.0000x reference)
import functools

import jax
import jax.numpy as jnp
from jax import lax
from jax.experimental import pallas as pl
from jax.experimental.pallas import tpu as pltpu
from jax.experimental.pallas import tpu_sc as plsc

N = 10000
E = 320000
D_E = 16
D_H = 256
NCLS = 8

NC = 2
NS = 16
LANE = 16

NP = 10112
RPT = NP // NS
EP = 327680
EPT = EP // NS
K = 80
NBLK = EPT // K
G = 4


C2 = 128
J_REGS = C2 // LANE


def _zero_accumulator(ebuf, aggs, s):
    def zrow(i, carry):
        for j in range(J_REGS):
            ebuf[i, pl.ds(j * LANE, LANE)] = jnp.zeros((LANE,), jnp.float32)
        return carry
    lax.fori_loop(0, K, zrow, None)
    r0 = s * RPT
    for t in range(RPT // K):
        pltpu.sync_copy(ebuf, aggs.at[pl.ds(r0 + t * K, K)])
    rem = RPT % K
    if rem:
        pltpu.sync_copy(ebuf.at[pl.ds(0, rem)],
                        aggs.at[pl.ds(r0 + (RPT // K) * K, rem)])
    plsc.subcore_barrier()
    return r0


def _sc_pipeline(xtab, es_slice, sdh, aggs, sbds, ebs, xgs, sems, gid0, nblk):
    sem_idx, sem_e, sem_g, sem_st = sems
    ngrp = nblk // G

    def idx_issue(sl, grp):
        pltpu.async_copy(sdh.at[pl.ds((gid0 + grp) * G, G)], sbds[sl], sem_idx)

    def idx_wait(sl):
        pltpu.make_async_copy(sdh.at[pl.ds(0, G)], sbds[sl], sem_idx).wait()

    def e_issue(sl, i):
        pltpu.async_copy(es_slice(i), ebs[sl], sem_e[sl])

    def e_wait(sl):
        pltpu.make_async_copy(es_slice(0), ebs[sl], sem_e[sl]).wait()

    def g_issue(sl, idxrow):
        pltpu.async_copy(xtab.at[idxrow], xgs[sl], sem_g[sl])

    def g_wait(sl):
        pltpu.make_async_copy(es_slice(0), xgs[sl], sem_g[sl]).wait()

    def st_wait():
        pltpu.make_async_copy(es_slice(0), aggs.at[pl.ds(0, K)], sem_st).wait()

    bid0 = gid0 * G
    H = K // 2

    def issue_ld(sl, i):
        pltpu.async_copy(sdh.at[bid0 + i], sbds[sl].at[0], sem_idx)
        pltpu.async_copy(es_slice(i), ebs[sl], sem_idx)

    def wait_ld(sl):
        pltpu.make_async_copy(sdh.at[0], sbds[sl].at[0], sem_idx).wait()
        pltpu.make_async_copy(es_slice(0), ebs[sl], sem_idx).wait()

    issue_ld(0, 0)

    def outer(g2, carry):
        for b in (0, 1):
            i = 2 * g2 + b
            sl = b
            wait_ld(sl)
            eb = ebs[sl]
            xg = xgs[0]
            cp = pltpu.async_copy(xtab.at[sbds[sl].at[0, 0]], xg, sem_g[0])

            @pl.when(i >= 1)
            def _():
                st_wait()

            @pl.when(i <= nblk - 2)
            def _():
                issue_ld(1 - sl, i + 1)

            cp.wait()

            def row(r, rcarry):
                for j in range(J_REGS):
                    v = pl.ds(j * LANE, LANE)
                    eb[r, v] = jnp.maximum(eb[r, v] + xg[r, v], 0.0)
                return rcarry
            lax.fori_loop(0, K, row, None)

            pltpu.async_copy(eb, aggs.at[sbds[sl].at[0, 1]], sem_st,
                             add=True)
        return carry

    lax.fori_loop(0, nblk // 2, outer, None)
    st_wait()


_SC_SCRATCH = [
    pltpu.VMEM((G, 2, K), jnp.int32),
    pltpu.VMEM((G, 2, K), jnp.int32),
    pltpu.VMEM((K, C2), jnp.float32),
    pltpu.VMEM((K, C2), jnp.float32),
    pltpu.VMEM((K, C2), jnp.float32),
    pltpu.VMEM((K, C2), jnp.float32),
    pltpu.VMEM_SHARED((NP, C2), jnp.float32),
    pltpu.SemaphoreType.DMA,
    pltpu.SemaphoreType.DMA,
    pltpu.SemaphoreType.DMA,
    pltpu.SemaphoreType.DMA,
    pltpu.SemaphoreType.DMA,
    pltpu.SemaphoreType.DMA,
]


def _sc_agg_edge():
    epw = EP // (NC * NS)
    nblk = epw // K
    mesh = plsc.VectorSubcoreMesh(core_axis_name="c", subcore_axis_name="s")

    @functools.partial(
        pl.kernel,
        out_type=jax.ShapeDtypeStruct((NC, NP, C2), jnp.float32),
        mesh=mesh,
        scratch_types=list(_SC_SCRATCH),
    )
    def k(xtab, es, sdh, out, sbd0, sbd1, eb0, eb1, xg0, xg1, aggs,
          sem_idx, se0, se1, sg0, sg1, sem_st):
        c = lax.axis_index("c")
        s = lax.axis_index("s")
        r0 = _zero_accumulator(eb0, aggs, s)
        ebase = (c * NS + s) * epw
        gid0 = (c * NS + s) * (nblk // G)
        _sc_pipeline(xtab, lambda i: es.at[pl.ds(ebase + i * K, K)], sdh,
                     aggs, (sbd0, sbd1), (eb0, eb1), (xg0, xg1),
                     (sem_idx, (se0, se1), (sg0, sg1), sem_st), gid0, nblk)
        plsc.subcore_barrier()
        pltpu.sync_copy(aggs.at[pl.ds(r0, RPT)], out.at[c, pl.ds(r0, RPT)])

    return k


def _sc_agg_feat():
    mesh = plsc.VectorSubcoreMesh(core_axis_name="c", subcore_axis_name="s")

    @functools.partial(
        pl.kernel,
        out_type=jax.ShapeDtypeStruct((NC, NP, C2), jnp.float32),
        mesh=mesh,
        scratch_types=list(_SC_SCRATCH),
    )
    def k(x0, x1, es, sdh, out, sbd0, sbd1, eb0, eb1, xg0, xg1, aggs,
          sem_idx, se0, se1, sg0, sg1, sem_st):
        c = lax.axis_index("c")
        s = lax.axis_index("s")
        r0 = _zero_accumulator(eb0, aggs, s)
        ebase = s * EPT
        gid0 = s * (NBLK // G)
        sems = (sem_idx, (se0, se1), (sg0, sg1), sem_st)

        @pl.when(c == 0)
        def _():
            _sc_pipeline(x0, lambda i: es.at[0, pl.ds(ebase + i * K, K)], sdh,
                         aggs, (sbd0, sbd1), (eb0, eb1), (xg0, xg1), sems,
                         gid0, NBLK)

        @pl.when(c == 1)
        def _():
            _sc_pipeline(x1, lambda i: es.at[1, pl.ds(ebase + i * K, K)], sdh,
                         aggs, (sbd0, sbd1), (eb0, eb1), (xg0, xg1), sems,
                         gid0, NBLK)

        plsc.subcore_barrier()
        pltpu.sync_copy(aggs.at[pl.ds(r0, RPT)], out.at[c, pl.ds(r0, RPT)])

    return k


_BE = 2048


def _e_matmul(cf):
    def body(a_ref, w_ref, b_ref, o_ref):
        h = jnp.dot(a_ref[...], w_ref[...], preferred_element_type=jnp.float32)
        h = h + b_ref[...]
        if cf == C2:
            o_ref[...] = h
        else:
            o_ref[0] = h[:, :C2]
            o_ref[1] = h[:, C2:]

    if cf == C2:
        out_spec = pl.BlockSpec((_BE, C2), lambda i: (i, 0))
        out_shape = jax.ShapeDtypeStruct((EP, C2), jnp.float32)
    else:
        out_spec = pl.BlockSpec((2, _BE, C2), lambda i: (0, i, 0))
        out_shape = jax.ShapeDtypeStruct((2, EP, C2), jnp.float32)

    return pl.pallas_call(
        body,
        grid=(EP // _BE,),
        in_specs=[
            pl.BlockSpec((_BE, D_E), lambda i: (i, 0)),
            pl.BlockSpec((D_E, cf), lambda i: (0, 0)),
            pl.BlockSpec((1, cf), lambda i: (0, 0)),
        ],
        out_specs=out_spec,
        out_shape=out_shape,
    )


_BN = 1000


def _mlp1(cf):
    def body(x_ref, ag_ref, w_ref, b_ref, eps_ref, h_ref, st_ref):
        i = pl.program_id(0)
        if cf == C2:
            agg = ag_ref[0] + ag_ref[1]
        else:
            agg = jnp.concatenate([ag_ref[0], ag_ref[1]], axis=1)
        out = eps_ref[0, 0] * x_ref[...] + agg
        h = jnp.dot(out, w_ref[...], preferred_element_type=jnp.float32) + b_ref[...]
        h_ref[...] = h

        @pl.when(i == 0)
        def _():
            st_ref[...] = jnp.zeros_like(st_ref)
        st_ref[...] += jnp.stack([jnp.sum(h, 0), jnp.sum(h * h, 0)], axis=0)

    return pl.pallas_call(
        body,
        grid=(N // _BN,),
        in_specs=[
            pl.BlockSpec((_BN, cf), lambda i: (i, 0)),
            pl.BlockSpec((2, _BN, C2), lambda i: (0, i, 0)),
            pl.BlockSpec((cf, D_H), lambda i: (0, 0)),
            pl.BlockSpec((1, D_H), lambda i: (0, 0)),
            pl.BlockSpec((1, 1), lambda i: (0, 0)),
        ],
        out_specs=[
            pl.BlockSpec((_BN, D_H), lambda i: (i, 0)),
            pl.BlockSpec((2, D_H), lambda i: (0, 0)),
        ],
        out_shape=[
            jax.ShapeDtypeStruct((N, D_H), jnp.float32),
            jax.ShapeDtypeStruct((2, D_H), jnp.float32),
        ],
    )


def _bn_stats(st_ref):
    mean = st_ref[0:1, :] * (1.0 / N)
    var = st_ref[1:2, :] * (1.0 / N) - mean * mean
    rstd = lax.rsqrt(var + 1e-5)
    return mean, rstd


def _lrelu(h):
    return jnp.where(h >= 0, h, 0.01 * h)


def _mlp2():
    def body(h1_ref, st_ref, g_ref, bt_ref, w_ref, b_ref, h_ref, st2_ref):
        i = pl.program_id(0)
        mean, rstd = _bn_stats(st_ref)
        hn = g_ref[...] * (h1_ref[...] - mean) * rstd + bt_ref[...]
        hn = _lrelu(hn)
        h = jnp.dot(hn, w_ref[...], preferred_element_type=jnp.float32) + b_ref[...]
        h_ref[...] = h

        @pl.when(i == 0)
        def _():
            st2_ref[...] = jnp.zeros_like(st2_ref)
        st2_ref[...] += jnp.stack([jnp.sum(h, 0), jnp.sum(h * h, 0)], axis=0)

    return pl.pallas_call(
        body,
        grid=(N // _BN,),
        in_specs=[
            pl.BlockSpec((_BN, D_H), lambda i: (i, 0)),
            pl.BlockSpec((2, D_H), lambda i: (0, 0)),
            pl.BlockSpec((1, D_H), lambda i: (0, 0)),
            pl.BlockSpec((1, D_H), lambda i: (0, 0)),
            pl.BlockSpec((D_H, D_H), lambda i: (0, 0)),
            pl.BlockSpec((1, D_H), lambda i: (0, 0)),
        ],
        out_specs=[
            pl.BlockSpec((_BN, D_H), lambda i: (i, 0)),
            pl.BlockSpec((2, D_H), lambda i: (0, 0)),
        ],
        out_shape=[
            jax.ShapeDtypeStruct((N, D_H), jnp.float32),
            jax.ShapeDtypeStruct((2, D_H), jnp.float32),
        ],
    )


def _mlp3():
    c2 = D_H // NC

    def body(h2_ref, st_ref, g_ref, b_ref, cur_ref, c0_ref, c1_ref):
        mean, rstd = _bn_stats(st_ref)
        v = g_ref[...] * (h2_ref[...] - mean) * rstd + b_ref[...]
        v = _lrelu(v)
        cur_ref[...] = v
        c0_ref[...] = v[:, :c2]
        c1_ref[...] = v[:, c2:]

    return pl.pallas_call(
        body,
        grid=(N // _BN,),
        in_specs=[
            pl.BlockSpec((_BN, D_H), lambda i: (i, 0)),
            pl.BlockSpec((2, D_H), lambda i: (0, 0)),
            pl.BlockSpec((1, D_H), lambda i: (0, 0)),
            pl.BlockSpec((1, D_H), lambda i: (0, 0)),
        ],
        out_specs=[
            pl.BlockSpec((_BN, D_H), lambda i: (i, 0)),
            pl.BlockSpec((_BN, c2), lambda i: (i, 0)),
            pl.BlockSpec((_BN, c2), lambda i: (i, 0)),
        ],
        out_shape=[
            jax.ShapeDtypeStruct((N, D_H), jnp.float32),
            jax.ShapeDtypeStruct((N, c2), jnp.float32),
            jax.ShapeDtypeStruct((N, c2), jnp.float32),
        ],
    )


def _final():
    def body(a_ref, b_ref, c_ref, w_ref, fb_ref, o_ref):
        h = jnp.concatenate([a_ref[...], b_ref[...], c_ref[...]], axis=1)
        logits = jnp.dot(h, w_ref[...], preferred_element_type=jnp.float32)
        logits = logits + fb_ref[...]
        col = lax.broadcasted_iota(jnp.int32, logits.shape, 1)
        logits = jnp.where(col < NCLS, logits, -1e30)
        m = jnp.max(logits, axis=1, keepdims=True)
        ex = jnp.exp(logits - m)
        o_ref[...] = ex / jnp.sum(ex, axis=1, keepdims=True)

    return pl.pallas_call(
        body,
        grid=(N // _BN,),
        in_specs=[
            pl.BlockSpec((_BN, D_H), lambda i: (i, 0)),
            pl.BlockSpec((_BN, D_H), lambda i: (i, 0)),
            pl.BlockSpec((_BN, D_H), lambda i: (i, 0)),
            pl.BlockSpec((3 * D_H, 128), lambda i: (0, 0)),
            pl.BlockSpec((1, 128), lambda i: (0, 0)),
        ],
        out_specs=pl.BlockSpec((_BN, 128), lambda i: (i, 0)),
        out_shape=jax.ShapeDtypeStruct((N, 128), jnp.float32),
    )


def kernel(x, edge_index, edge_attr, params):
    src = edge_index[0]
    dst = edge_index[1]
    pad = EP - E
    srcp = jnp.concatenate([src, jnp.zeros((pad,), jnp.int32)])
    dstp = jnp.concatenate([dst, jnp.full((pad,), N, jnp.int32)])
    sdh = jnp.stack([srcp.reshape(-1, K), dstp.reshape(-1, K)], axis=1)
    eap = jnp.concatenate([edge_attr, jnp.zeros((pad, D_E), jnp.float32)], axis=0)

    cur = x
    x0 = x1 = None
    hs = []
    for p in params["layers"]:
        cf = cur.shape[1]
        es = _e_matmul(cf)(eap, p["lin_e_w"], p["lin_e_b"].reshape(1, -1))
        if cf == C2:
            agg = _sc_agg_edge()(cur, es, sdh)
        else:
            agg = _sc_agg_feat()(x0, x1, es, sdh)
        h1, st1 = _mlp1(cf)(
            cur, agg, p["w1"], p["b1"].reshape(1, -1),
            (1.0 + p["eps"]).reshape(1, 1),
        )
        h2, st2 = _mlp2()(
            h1, st1, p["g1"].reshape(1, -1), p["bt1"].reshape(1, -1),
            p["w2"], p["b2"].reshape(1, -1),
        )
        cur, x0, x1 = _mlp3()(
            h2, st2, p["g_out"].reshape(1, -1), p["b_out"].reshape(1, -1)
        )
        hs.append(cur)

    fw = jnp.concatenate(
        [params["final_w"],
         jnp.zeros((3 * D_H, 128 - NCLS), jnp.float32)], axis=1)
    fb = jnp.concatenate(
        [params["final_b"], jnp.zeros((128 - NCLS,), jnp.float32)]).reshape(1, -1)
    probs = _final()(hs[0], hs[1], hs[2], fw, fb)
    return probs[:, :NCLS]


# speedup vs baseline: 1.0269x; 1.0269x over previous
import functools

import jax
import jax.numpy as jnp
from jax import lax
from jax.experimental import pallas as pl
from jax.experimental.pallas import tpu as pltpu
from jax.experimental.pallas import tpu_sc as plsc

N = 10000
E = 320000
D_E = 16
D_H = 256
NCLS = 8

NC = 2
NS = 16
LANE = 16

NP = 10112
RPT = NP // NS
EP = 327680
EPT = EP // NS
K = 128
NBLK = EPT // K


C2 = 128
J_REGS = C2 // LANE


def _zero_accumulator(ebuf, aggs, s):
    def zrow(i, carry):
        for j in range(J_REGS):
            ebuf[i, pl.ds(j * LANE, LANE)] = jnp.zeros((LANE,), jnp.float32)
        return carry
    lax.fori_loop(0, K, zrow, None)
    r0 = s * RPT
    for t in range(RPT // K):
        pltpu.sync_copy(ebuf, aggs.at[pl.ds(r0 + t * K, K)])
    rem = RPT % K
    if rem:
        pltpu.sync_copy(ebuf.at[pl.ds(0, rem)],
                        aggs.at[pl.ds(r0 + (RPT // K) * K, rem)])
    plsc.subcore_barrier()
    return r0


def _sc_pipeline(xtab, es_slice, sdh, aggs, sbds, ebs, xg, sems, bid0, nblk):
    sem_ld, sem_g, sem_st = sems

    def issue_ld(sl, i):
        pltpu.async_copy(sdh.at[bid0 + i], sbds[sl].at[0], sem_ld)
        pltpu.async_copy(es_slice(i), ebs[sl], sem_ld)

    def wait_ld(sl):
        pltpu.make_async_copy(sdh.at[0], sbds[sl].at[0], sem_ld).wait()
        pltpu.make_async_copy(es_slice(0), ebs[sl], sem_ld).wait()

    def st_wait():
        pltpu.make_async_copy(es_slice(0), aggs.at[pl.ds(0, K)], sem_st).wait()

    issue_ld(0, 0)

    def outer(g2, carry):
        for b in (0, 1):
            i = 2 * g2 + b
            sl = b
            wait_ld(sl)
            eb = ebs[sl]
            cp = pltpu.async_copy(xtab.at[sbds[sl].at[0, 0]], xg, sem_g)

            @pl.when(i >= 1)
            def _():
                st_wait()

            @pl.when(i <= nblk - 2)
            def _():
                issue_ld(1 - sl, i + 1)

            cp.wait()

            def row(r, rcarry):
                for j in range(J_REGS):
                    v = pl.ds(j * LANE, LANE)
                    eb[r, v] = jnp.maximum(eb[r, v] + xg[r, v], 0.0)
                return rcarry
            lax.fori_loop(0, K, row, None)

            pltpu.async_copy(eb, aggs.at[sbds[sl].at[0, 1]], sem_st,
                             add=True)
        return carry

    lax.fori_loop(0, nblk // 2, outer, None)
    st_wait()


_SC_SCRATCH = [
    pltpu.VMEM((1, 2, K), jnp.int32),
    pltpu.VMEM((1, 2, K), jnp.int32),
    pltpu.VMEM((K, C2), jnp.float32),
    pltpu.VMEM((K, C2), jnp.float32),
    pltpu.VMEM((K, C2), jnp.float32),
    pltpu.VMEM_SHARED((NP, C2), jnp.float32),
    pltpu.SemaphoreType.DMA,
    pltpu.SemaphoreType.DMA,
    pltpu.SemaphoreType.DMA,
]


def _sc_agg_edge():
    epw = EP // (NC * NS)
    nblk = epw // K
    mesh = plsc.VectorSubcoreMesh(core_axis_name="c", subcore_axis_name="s")

    @functools.partial(
        pl.kernel,
        out_type=jax.ShapeDtypeStruct((NC, NP, C2), jnp.float32),
        mesh=mesh,
        scratch_types=list(_SC_SCRATCH),
    )
    def k(xtab, es, sdh, out, sbd0, sbd1, eb0, eb1, xg, aggs,
          sem_ld, sem_g, sem_st):
        c = lax.axis_index("c")
        s = lax.axis_index("s")
        r0 = _zero_accumulator(eb0, aggs, s)
        ebase = (c * NS + s) * epw
        bid0 = (c * NS + s) * nblk
        _sc_pipeline(xtab, lambda i: es.at[pl.ds(ebase + i * K, K)], sdh,
                     aggs, (sbd0, sbd1), (eb0, eb1), xg,
                     (sem_ld, sem_g, sem_st), bid0, nblk)
        plsc.subcore_barrier()
        pltpu.sync_copy(aggs.at[pl.ds(r0, RPT)], out.at[c, pl.ds(r0, RPT)])

    return k


def _sc_agg_feat():
    mesh = plsc.VectorSubcoreMesh(core_axis_name="c", subcore_axis_name="s")

    @functools.partial(
        pl.kernel,
        out_type=jax.ShapeDtypeStruct((NC, NP, C2), jnp.float32),
        mesh=mesh,
        scratch_types=list(_SC_SCRATCH),
    )
    def k(x0, x1, es, sdh, out, sbd0, sbd1, eb0, eb1, xg, aggs,
          sem_ld, sem_g, sem_st):
        c = lax.axis_index("c")
        s = lax.axis_index("s")
        r0 = _zero_accumulator(eb0, aggs, s)
        ebase = s * EPT
        bid0 = s * NBLK
        sems = (sem_ld, sem_g, sem_st)

        @pl.when(c == 0)
        def _():
            _sc_pipeline(x0, lambda i: es.at[0, pl.ds(ebase + i * K, K)], sdh,
                         aggs, (sbd0, sbd1), (eb0, eb1), xg, sems,
                         bid0, NBLK)

        @pl.when(c == 1)
        def _():
            _sc_pipeline(x1, lambda i: es.at[1, pl.ds(ebase + i * K, K)], sdh,
                         aggs, (sbd0, sbd1), (eb0, eb1), xg, sems,
                         bid0, NBLK)

        plsc.subcore_barrier()
        pltpu.sync_copy(aggs.at[pl.ds(r0, RPT)], out.at[c, pl.ds(r0, RPT)])

    return k


_BE = 2048


def _e_matmul(cf):
    def body(a_ref, w_ref, b_ref, o_ref):
        h = jnp.dot(a_ref[...], w_ref[...], preferred_element_type=jnp.float32)
        h = h + b_ref[...]
        if cf == C2:
            o_ref[...] = h
        else:
            o_ref[0] = h[:, :C2]
            o_ref[1] = h[:, C2:]

    if cf == C2:
        out_spec = pl.BlockSpec((_BE, C2), lambda i: (i, 0))
        out_shape = jax.ShapeDtypeStruct((EP, C2), jnp.float32)
    else:
        out_spec = pl.BlockSpec((2, _BE, C2), lambda i: (0, i, 0))
        out_shape = jax.ShapeDtypeStruct((2, EP, C2), jnp.float32)

    return pl.pallas_call(
        body,
        grid=(EP // _BE,),
        in_specs=[
            pl.BlockSpec((_BE, D_E), lambda i: (i, 0)),
            pl.BlockSpec((D_E, cf), lambda i: (0, 0)),
            pl.BlockSpec((1, cf), lambda i: (0, 0)),
        ],
        out_specs=out_spec,
        out_shape=out_shape,
    )


_BN = 1000


def _mlp1(cf):
    def body(x_ref, ag_ref, w_ref, b_ref, eps_ref, h_ref, st_ref):
        i = pl.program_id(0)
        if cf == C2:
            agg = ag_ref[0] + ag_ref[1]
        else:
            agg = jnp.concatenate([ag_ref[0], ag_ref[1]], axis=1)
        out = eps_ref[0, 0] * x_ref[...] + agg
        h = jnp.dot(out, w_ref[...], preferred_element_type=jnp.float32) + b_ref[...]
        h_ref[...] = h

        @pl.when(i == 0)
        def _():
            st_ref[...] = jnp.zeros_like(st_ref)
        st_ref[...] += jnp.stack([jnp.sum(h, 0), jnp.sum(h * h, 0)], axis=0)

    return pl.pallas_call(
        body,
        grid=(N // _BN,),
        in_specs=[
            pl.BlockSpec((_BN, cf), lambda i: (i, 0)),
            pl.BlockSpec((2, _BN, C2), lambda i: (0, i, 0)),
            pl.BlockSpec((cf, D_H), lambda i: (0, 0)),
            pl.BlockSpec((1, D_H), lambda i: (0, 0)),
            pl.BlockSpec((1, 1), lambda i: (0, 0)),
        ],
        out_specs=[
            pl.BlockSpec((_BN, D_H), lambda i: (i, 0)),
            pl.BlockSpec((2, D_H), lambda i: (0, 0)),
        ],
        out_shape=[
            jax.ShapeDtypeStruct((N, D_H), jnp.float32),
            jax.ShapeDtypeStruct((2, D_H), jnp.float32),
        ],
    )


def _bn_stats(st_ref):
    mean = st_ref[0:1, :] * (1.0 / N)
    var = st_ref[1:2, :] * (1.0 / N) - mean * mean
    rstd = lax.rsqrt(var + 1e-5)
    return mean, rstd


def _lrelu(h):
    return jnp.where(h >= 0, h, 0.01 * h)


def _mlp2():
    def body(h1_ref, st_ref, g_ref, bt_ref, w_ref, b_ref, h_ref, st2_ref):
        i = pl.program_id(0)
        mean, rstd = _bn_stats(st_ref)
        hn = g_ref[...] * (h1_ref[...] - mean) * rstd + bt_ref[...]
        hn = _lrelu(hn)
        h = jnp.dot(hn, w_ref[...], preferred_element_type=jnp.float32) + b_ref[...]
        h_ref[...] = h

        @pl.when(i == 0)
        def _():
            st2_ref[...] = jnp.zeros_like(st2_ref)
        st2_ref[...] += jnp.stack([jnp.sum(h, 0), jnp.sum(h * h, 0)], axis=0)

    return pl.pallas_call(
        body,
        grid=(N // _BN,),
        in_specs=[
            pl.BlockSpec((_BN, D_H), lambda i: (i, 0)),
            pl.BlockSpec((2, D_H), lambda i: (0, 0)),
            pl.BlockSpec((1, D_H), lambda i: (0, 0)),
            pl.BlockSpec((1, D_H), lambda i: (0, 0)),
            pl.BlockSpec((D_H, D_H), lambda i: (0, 0)),
            pl.BlockSpec((1, D_H), lambda i: (0, 0)),
        ],
        out_specs=[
            pl.BlockSpec((_BN, D_H), lambda i: (i, 0)),
            pl.BlockSpec((2, D_H), lambda i: (0, 0)),
        ],
        out_shape=[
            jax.ShapeDtypeStruct((N, D_H), jnp.float32),
            jax.ShapeDtypeStruct((2, D_H), jnp.float32),
        ],
    )


def _mlp3():
    c2 = D_H // NC

    def body(h2_ref, st_ref, g_ref, b_ref, cur_ref, c0_ref, c1_ref):
        mean, rstd = _bn_stats(st_ref)
        v = g_ref[...] * (h2_ref[...] - mean) * rstd + b_ref[...]
        v = _lrelu(v)
        cur_ref[...] = v
        c0_ref[...] = v[:, :c2]
        c1_ref[...] = v[:, c2:]

    return pl.pallas_call(
        body,
        grid=(N // _BN,),
        in_specs=[
            pl.BlockSpec((_BN, D_H), lambda i: (i, 0)),
            pl.BlockSpec((2, D_H), lambda i: (0, 0)),
            pl.BlockSpec((1, D_H), lambda i: (0, 0)),
            pl.BlockSpec((1, D_H), lambda i: (0, 0)),
        ],
        out_specs=[
            pl.BlockSpec((_BN, D_H), lambda i: (i, 0)),
            pl.BlockSpec((_BN, c2), lambda i: (i, 0)),
            pl.BlockSpec((_BN, c2), lambda i: (i, 0)),
        ],
        out_shape=[
            jax.ShapeDtypeStruct((N, D_H), jnp.float32),
            jax.ShapeDtypeStruct((N, c2), jnp.float32),
            jax.ShapeDtypeStruct((N, c2), jnp.float32),
        ],
    )


def _final():
    def body(a_ref, b_ref, c_ref, w_ref, fb_ref, o_ref):
        h = jnp.concatenate([a_ref[...], b_ref[...], c_ref[...]], axis=1)
        logits = jnp.dot(h, w_ref[...], preferred_element_type=jnp.float32)
        logits = logits + fb_ref[...]
        col = lax.broadcasted_iota(jnp.int32, logits.shape, 1)
        logits = jnp.where(col < NCLS, logits, -1e30)
        m = jnp.max(logits, axis=1, keepdims=True)
        ex = jnp.exp(logits - m)
        o_ref[...] = ex / jnp.sum(ex, axis=1, keepdims=True)

    return pl.pallas_call(
        body,
        grid=(N // _BN,),
        in_specs=[
            pl.BlockSpec((_BN, D_H), lambda i: (i, 0)),
            pl.BlockSpec((_BN, D_H), lambda i: (i, 0)),
            pl.BlockSpec((_BN, D_H), lambda i: (i, 0)),
            pl.BlockSpec((3 * D_H, 128), lambda i: (0, 0)),
            pl.BlockSpec((1, 128), lambda i: (0, 0)),
        ],
        out_specs=pl.BlockSpec((_BN, 128), lambda i: (i, 0)),
        out_shape=jax.ShapeDtypeStruct((N, 128), jnp.float32),
    )


def kernel(x, edge_index, edge_attr, params):
    src = edge_index[0]
    dst = edge_index[1]
    pad = EP - E
    srcp = jnp.concatenate([src, jnp.zeros((pad,), jnp.int32)])
    dstp = jnp.concatenate([dst, jnp.full((pad,), N, jnp.int32)])
    sdh = jnp.stack([srcp.reshape(-1, K), dstp.reshape(-1, K)], axis=1)
    eap = jnp.concatenate([edge_attr, jnp.zeros((pad, D_E), jnp.float32)], axis=0)

    cur = x
    x0 = x1 = None
    hs = []
    for p in params["layers"]:
        cf = cur.shape[1]
        es = _e_matmul(cf)(eap, p["lin_e_w"], p["lin_e_b"].reshape(1, -1))
        if cf == C2:
            agg = _sc_agg_edge()(cur, es, sdh)
        else:
            agg = _sc_agg_feat()(x0, x1, es, sdh)
        h1, st1 = _mlp1(cf)(
            cur, agg, p["w1"], p["b1"].reshape(1, -1),
            (1.0 + p["eps"]).reshape(1, 1),
        )
        h2, st2 = _mlp2()(
            h1, st1, p["g1"].reshape(1, -1), p["bt1"].reshape(1, -1),
            p["w2"], p["b2"].reshape(1, -1),
        )
        cur, x0, x1 = _mlp3()(
            h2, st2, p["g_out"].reshape(1, -1), p["b_out"].reshape(1, -1)
        )
        hs.append(cur)

    fw = jnp.concatenate(
        [params["final_w"],
         jnp.zeros((3 * D_H, 128 - NCLS), jnp.float32)], axis=1)
    fb = jnp.concatenate(
        [params["final_b"], jnp.zeros((128 - NCLS,), jnp.float32)]).reshape(1, -1)
    probs = _final()(hs[0], hs[1], hs[2], fw, fb)
    return probs[:, :NCLS]


# speedup vs baseline: 1.0528x; 1.0252x over previous
import functools

import jax
import jax.numpy as jnp
from jax import lax
from jax.experimental import pallas as pl
from jax.experimental.pallas import tpu as pltpu
from jax.experimental.pallas import tpu_sc as plsc

N = 10000
E = 320000
D_E = 16
D_H = 256
NCLS = 8

NC = 2
NS = 16
LANE = 16

NP = 10112
RPT = NP // NS
EP = 327680
EPT = EP // NS
K = 128
NBLK = EPT // K


C2 = 128
J_REGS = C2 // LANE


def _zero_accumulator(ebuf, aggs, s):
    def zrow(i, carry):
        for j in range(J_REGS):
            ebuf[i, pl.ds(j * LANE, LANE)] = jnp.zeros((LANE,), jnp.float32)
        return carry
    lax.fori_loop(0, K, zrow, None)
    r0 = s * RPT
    for t in range(RPT // K):
        pltpu.sync_copy(ebuf, aggs.at[pl.ds(r0 + t * K, K)])
    rem = RPT % K
    if rem:
        pltpu.sync_copy(ebuf.at[pl.ds(0, rem)],
                        aggs.at[pl.ds(r0 + (RPT // K) * K, rem)])
    plsc.subcore_barrier()
    return r0


def _sc_pipeline(xtab, es_slice, sdh, aggs, sbds, ebs, xg, sems, bid0, nblk):
    sem_ld, sem_g, sem_st = sems

    def issue_ld(sl, i):
        pltpu.async_copy(sdh.at[bid0 + i], sbds[sl].at[0], sem_ld)
        pltpu.async_copy(es_slice(i), ebs[sl], sem_ld)

    def wait_ld(sl):
        pltpu.make_async_copy(sdh.at[0], sbds[sl].at[0], sem_ld).wait()
        pltpu.make_async_copy(es_slice(0), ebs[sl], sem_ld).wait()

    def st_wait():
        pltpu.make_async_copy(es_slice(0), aggs.at[pl.ds(0, K)], sem_st).wait()

    issue_ld(0, 0)

    def outer(g2, carry):
        for b in (0, 1):
            i = 2 * g2 + b
            sl = b
            wait_ld(sl)
            eb = ebs[sl]
            pltpu.async_copy(xtab.at[sbds[sl].at[0, 0]], xg, sem_g).wait()

            @pl.when(i >= 1)
            def _():
                st_wait()

            @pl.when(i <= nblk - 2)
            def _():
                issue_ld(1 - sl, i + 1)

            def row(r, rcarry):
                for j in range(J_REGS):
                    v = pl.ds(j * LANE, LANE)
                    eb[r, v] = jnp.maximum(eb[r, v] + xg[r, v], 0.0)
                return rcarry
            lax.fori_loop(0, K, row, None)

            pltpu.async_copy(eb, aggs.at[sbds[sl].at[0, 1]], sem_st,
                             add=True)
        return carry

    lax.fori_loop(0, nblk // 2, outer, None)
    st_wait()


_SC_SCRATCH = [
    pltpu.VMEM((1, 2, K), jnp.int32),
    pltpu.VMEM((1, 2, K), jnp.int32),
    pltpu.VMEM((K, C2), jnp.float32),
    pltpu.VMEM((K, C2), jnp.float32),
    pltpu.VMEM((K, C2), jnp.float32),
    pltpu.VMEM_SHARED((NP, C2), jnp.float32),
    pltpu.SemaphoreType.DMA,
    pltpu.SemaphoreType.DMA,
    pltpu.SemaphoreType.DMA,
]


def _sc_agg_edge():
    epw = EP // (NC * NS)
    nblk = epw // K
    mesh = plsc.VectorSubcoreMesh(core_axis_name="c", subcore_axis_name="s")

    @functools.partial(
        pl.kernel,
        out_type=jax.ShapeDtypeStruct((NC, NP, C2), jnp.float32),
        mesh=mesh,
        scratch_types=list(_SC_SCRATCH),
    )
    def k(xtab, es, sdh, out, sbd0, sbd1, eb0, eb1, xg, aggs,
          sem_ld, sem_g, sem_st):
        c = lax.axis_index("c")
        s = lax.axis_index("s")
        r0 = _zero_accumulator(eb0, aggs, s)
        ebase = (c * NS + s) * epw
        bid0 = (c * NS + s) * nblk
        _sc_pipeline(xtab, lambda i: es.at[pl.ds(ebase + i * K, K)], sdh,
                     aggs, (sbd0, sbd1), (eb0, eb1), xg,
                     (sem_ld, sem_g, sem_st), bid0, nblk)
        plsc.subcore_barrier()
        pltpu.sync_copy(aggs.at[pl.ds(r0, RPT)], out.at[c, pl.ds(r0, RPT)])

    return k


def _sc_agg_feat():
    mesh = plsc.VectorSubcoreMesh(core_axis_name="c", subcore_axis_name="s")

    @functools.partial(
        pl.kernel,
        out_type=jax.ShapeDtypeStruct((NC, NP, C2), jnp.float32),
        mesh=mesh,
        scratch_types=list(_SC_SCRATCH),
    )
    def k(x0, x1, es, sdh, out, sbd0, sbd1, eb0, eb1, xg, aggs,
          sem_ld, sem_g, sem_st):
        c = lax.axis_index("c")
        s = lax.axis_index("s")
        r0 = _zero_accumulator(eb0, aggs, s)
        ebase = s * EPT
        bid0 = s * NBLK
        sems = (sem_ld, sem_g, sem_st)

        @pl.when(c == 0)
        def _():
            _sc_pipeline(x0, lambda i: es.at[0, pl.ds(ebase + i * K, K)], sdh,
                         aggs, (sbd0, sbd1), (eb0, eb1), xg, sems,
                         bid0, NBLK)

        @pl.when(c == 1)
        def _():
            _sc_pipeline(x1, lambda i: es.at[1, pl.ds(ebase + i * K, K)], sdh,
                         aggs, (sbd0, sbd1), (eb0, eb1), xg, sems,
                         bid0, NBLK)

        plsc.subcore_barrier()
        pltpu.sync_copy(aggs.at[pl.ds(r0, RPT)], out.at[c, pl.ds(r0, RPT)])

    return k


_BE = 2048


def _e_matmul(cf):
    def body(a_ref, w_ref, b_ref, o_ref):
        h = jnp.dot(a_ref[...], w_ref[...], preferred_element_type=jnp.float32)
        h = h + b_ref[...]
        if cf == C2:
            o_ref[...] = h
        else:
            o_ref[0] = h[:, :C2]
            o_ref[1] = h[:, C2:]

    if cf == C2:
        out_spec = pl.BlockSpec((_BE, C2), lambda i: (i, 0))
        out_shape = jax.ShapeDtypeStruct((EP, C2), jnp.float32)
    else:
        out_spec = pl.BlockSpec((2, _BE, C2), lambda i: (0, i, 0))
        out_shape = jax.ShapeDtypeStruct((2, EP, C2), jnp.float32)

    return pl.pallas_call(
        body,
        grid=(EP // _BE,),
        in_specs=[
            pl.BlockSpec((_BE, D_E), lambda i: (i, 0)),
            pl.BlockSpec((D_E, cf), lambda i: (0, 0)),
            pl.BlockSpec((1, cf), lambda i: (0, 0)),
        ],
        out_specs=out_spec,
        out_shape=out_shape,
    )


_BN = 1000


def _mlp1(cf):
    def body(x_ref, ag_ref, w_ref, b_ref, eps_ref, h_ref, st_ref):
        i = pl.program_id(0)
        if cf == C2:
            agg = ag_ref[0] + ag_ref[1]
        else:
            agg = jnp.concatenate([ag_ref[0], ag_ref[1]], axis=1)
        out = eps_ref[0, 0] * x_ref[...] + agg
        h = jnp.dot(out, w_ref[...], preferred_element_type=jnp.float32) + b_ref[...]
        h_ref[...] = h

        @pl.when(i == 0)
        def _():
            st_ref[...] = jnp.zeros_like(st_ref)
        st_ref[...] += jnp.stack([jnp.sum(h, 0), jnp.sum(h * h, 0)], axis=0)

    return pl.pallas_call(
        body,
        grid=(N // _BN,),
        in_specs=[
            pl.BlockSpec((_BN, cf), lambda i: (i, 0)),
            pl.BlockSpec((2, _BN, C2), lambda i: (0, i, 0)),
            pl.BlockSpec((cf, D_H), lambda i: (0, 0)),
            pl.BlockSpec((1, D_H), lambda i: (0, 0)),
            pl.BlockSpec((1, 1), lambda i: (0, 0)),
        ],
        out_specs=[
            pl.BlockSpec((_BN, D_H), lambda i: (i, 0)),
            pl.BlockSpec((2, D_H), lambda i: (0, 0)),
        ],
        out_shape=[
            jax.ShapeDtypeStruct((N, D_H), jnp.float32),
            jax.ShapeDtypeStruct((2, D_H), jnp.float32),
        ],
    )


def _bn_stats(st_ref):
    mean = st_ref[0:1, :] * (1.0 / N)
    var = st_ref[1:2, :] * (1.0 / N) - mean * mean
    rstd = lax.rsqrt(var + 1e-5)
    return mean, rstd


def _lrelu(h):
    return jnp.where(h >= 0, h, 0.01 * h)


def _mlp2():
    def body(h1_ref, st_ref, g_ref, bt_ref, w_ref, b_ref, h_ref, st2_ref):
        i = pl.program_id(0)
        mean, rstd = _bn_stats(st_ref)
        hn = g_ref[...] * (h1_ref[...] - mean) * rstd + bt_ref[...]
        hn = _lrelu(hn)
        h = jnp.dot(hn, w_ref[...], preferred_element_type=jnp.float32) + b_ref[...]
        h_ref[...] = h

        @pl.when(i == 0)
        def _():
            st2_ref[...] = jnp.zeros_like(st2_ref)
        st2_ref[...] += jnp.stack([jnp.sum(h, 0), jnp.sum(h * h, 0)], axis=0)

    return pl.pallas_call(
        body,
        grid=(N // _BN,),
        in_specs=[
            pl.BlockSpec((_BN, D_H), lambda i: (i, 0)),
            pl.BlockSpec((2, D_H), lambda i: (0, 0)),
            pl.BlockSpec((1, D_H), lambda i: (0, 0)),
            pl.BlockSpec((1, D_H), lambda i: (0, 0)),
            pl.BlockSpec((D_H, D_H), lambda i: (0, 0)),
            pl.BlockSpec((1, D_H), lambda i: (0, 0)),
        ],
        out_specs=[
            pl.BlockSpec((_BN, D_H), lambda i: (i, 0)),
            pl.BlockSpec((2, D_H), lambda i: (0, 0)),
        ],
        out_shape=[
            jax.ShapeDtypeStruct((N, D_H), jnp.float32),
            jax.ShapeDtypeStruct((2, D_H), jnp.float32),
        ],
    )


def _mlp3():
    c2 = D_H // NC

    def body(h2_ref, st_ref, g_ref, b_ref, cur_ref, c0_ref, c1_ref):
        mean, rstd = _bn_stats(st_ref)
        v = g_ref[...] * (h2_ref[...] - mean) * rstd + b_ref[...]
        v = _lrelu(v)
        cur_ref[...] = v
        c0_ref[...] = v[:, :c2]
        c1_ref[...] = v[:, c2:]

    return pl.pallas_call(
        body,
        grid=(N // _BN,),
        in_specs=[
            pl.BlockSpec((_BN, D_H), lambda i: (i, 0)),
            pl.BlockSpec((2, D_H), lambda i: (0, 0)),
            pl.BlockSpec((1, D_H), lambda i: (0, 0)),
            pl.BlockSpec((1, D_H), lambda i: (0, 0)),
        ],
        out_specs=[
            pl.BlockSpec((_BN, D_H), lambda i: (i, 0)),
            pl.BlockSpec((_BN, c2), lambda i: (i, 0)),
            pl.BlockSpec((_BN, c2), lambda i: (i, 0)),
        ],
        out_shape=[
            jax.ShapeDtypeStruct((N, D_H), jnp.float32),
            jax.ShapeDtypeStruct((N, c2), jnp.float32),
            jax.ShapeDtypeStruct((N, c2), jnp.float32),
        ],
    )


def _final():
    def body(a_ref, b_ref, c_ref, w_ref, fb_ref, o_ref):
        h = jnp.concatenate([a_ref[...], b_ref[...], c_ref[...]], axis=1)
        logits = jnp.dot(h, w_ref[...], preferred_element_type=jnp.float32)
        logits = logits + fb_ref[...]
        col = lax.broadcasted_iota(jnp.int32, logits.shape, 1)
        logits = jnp.where(col < NCLS, logits, -1e30)
        m = jnp.max(logits, axis=1, keepdims=True)
        ex = jnp.exp(logits - m)
        o_ref[...] = ex / jnp.sum(ex, axis=1, keepdims=True)

    return pl.pallas_call(
        body,
        grid=(N // _BN,),
        in_specs=[
            pl.BlockSpec((_BN, D_H), lambda i: (i, 0)),
            pl.BlockSpec((_BN, D_H), lambda i: (i, 0)),
            pl.BlockSpec((_BN, D_H), lambda i: (i, 0)),
            pl.BlockSpec((3 * D_H, 128), lambda i: (0, 0)),
            pl.BlockSpec((1, 128), lambda i: (0, 0)),
        ],
        out_specs=pl.BlockSpec((_BN, 128), lambda i: (i, 0)),
        out_shape=jax.ShapeDtypeStruct((N, 128), jnp.float32),
    )


def kernel(x, edge_index, edge_attr, params):
    src = edge_index[0]
    dst = edge_index[1]
    pad = EP - E
    srcp = jnp.concatenate([src, jnp.zeros((pad,), jnp.int32)])
    dstp = jnp.concatenate([dst, jnp.full((pad,), N, jnp.int32)])
    sdh = jnp.stack([srcp.reshape(-1, K), dstp.reshape(-1, K)], axis=1)
    eap = jnp.concatenate([edge_attr, jnp.zeros((pad, D_E), jnp.float32)], axis=0)

    cur = x
    x0 = x1 = None
    hs = []
    for p in params["layers"]:
        cf = cur.shape[1]
        es = _e_matmul(cf)(eap, p["lin_e_w"], p["lin_e_b"].reshape(1, -1))
        if cf == C2:
            agg = _sc_agg_edge()(cur, es, sdh)
        else:
            agg = _sc_agg_feat()(x0, x1, es, sdh)
        h1, st1 = _mlp1(cf)(
            cur, agg, p["w1"], p["b1"].reshape(1, -1),
            (1.0 + p["eps"]).reshape(1, 1),
        )
        h2, st2 = _mlp2()(
            h1, st1, p["g1"].reshape(1, -1), p["bt1"].reshape(1, -1),
            p["w2"], p["b2"].reshape(1, -1),
        )
        cur, x0, x1 = _mlp3()(
            h2, st2, p["g_out"].reshape(1, -1), p["b_out"].reshape(1, -1)
        )
        hs.append(cur)

    fw = jnp.concatenate(
        [params["final_w"],
         jnp.zeros((3 * D_H, 128 - NCLS), jnp.float32)], axis=1)
    fb = jnp.concatenate(
        [params["final_b"], jnp.zeros((128 - NCLS,), jnp.float32)]).reshape(1, -1)
    probs = _final()(hs[0], hs[1], hs[2], fw, fb)
    return probs[:, :NCLS]
